# Initial kernel scaffold; baseline (speedup 1.0000x reference)
import jax
import jax.numpy as jnp
from jax.experimental import pallas as pl


def kernel(inputs, unnormalized_widths, unnormalized_heights, unnorm_derivatives_left, unnorm_derivatives_right):
    raise NotImplementedError("write your pallas kernel here")


# speedup vs baseline: 1361.4076x; 1361.4076x over previous
import functools

import jax
import jax.numpy as jnp
from jax import lax
from jax.experimental import pallas as pl

_MIN_W = 1e-3
_MIN_H = 1e-3
_NB = 32


def _softmax0(x):
    m = jnp.max(x, axis=0, keepdims=True)
    e = jnp.exp(x - m)
    return e / jnp.sum(e, axis=0, keepdims=True)


def _prep_tables(uwt, uht, udlt, udrt):
    nb = _NB
    w = _softmax0(uwt)
    w = _MIN_W + (1.0 - _MIN_W * nb) * w
    h = _softmax0(uht)
    h = _MIN_H + (1.0 - _MIN_H * nb) * h

    row = lax.broadcasted_iota(jnp.int32, (nb, nb), 0)
    col = lax.broadcasted_iota(jnp.int32, (nb, nb), 1)
    tri = (col <= row).astype(jnp.float32)
    cums_w = jnp.dot(tri, w, preferred_element_type=jnp.float32)
    cums_h = jnp.dot(tri, h, preferred_element_type=jnp.float32)

    zero = jnp.zeros_like(w[0:1])
    cw = jnp.concatenate([zero, cums_w[: nb - 1]], axis=0)
    dtab = jnp.concatenate([zero, cums_h[: nb - 1]], axis=0)

    s = h / w
    s_lo, s_hi = s[: nb - 1], s[1:]
    w_lo, w_hi = w[: nb - 1], w[1:]
    min1 = jnp.minimum(jnp.abs(s_lo), jnp.abs(s_hi))
    min2 = 0.5 * (w_hi * s_lo + w_lo * s_hi) / (w_lo + w_hi)
    dmid = jnp.minimum(min1, min2) * (jnp.sign(s_lo) + jnp.sign(s_hi))

    d0 = jax.nn.sigmoid(udlt) * 3.0 * s[0:1]
    dN = jax.nn.sigmoid(udrt) * 3.0 * s[nb - 1 : nb]
    dlo = jnp.concatenate([d0, dmid], axis=0)
    dhi = jnp.concatenate([dmid, dN], axis=0)

    a = (dlo + dhi - 2.0 * s) / (w * w)
    b = (3.0 * s - 2.0 * dlo - dhi) / w
    c = dlo
    return cw, a, b, c, dtab


def _body(x_ref, uwt_ref, uht_ref, udlt_ref, udrt_ref, out_ref, lad_ref):
    cw, a, b, c, d = _prep_tables(
        uwt_ref[...], uht_ref[...], udlt_ref[...], udrt_ref[...]
    )
    x = x_ref[...]
    sh = x.shape
    acc_a = jnp.broadcast_to(a[0:1], sh)
    acc_b = jnp.broadcast_to(b[0:1], sh)
    acc_c = jnp.broadcast_to(c[0:1], sh)
    acc_d = jnp.broadcast_to(d[0:1], sh)
    acc_w = jnp.zeros(sh, jnp.float32)
    for k in range(1, _NB):
        m = x >= cw[k : k + 1]
        acc_a = jnp.where(m, a[k : k + 1], acc_a)
        acc_b = jnp.where(m, b[k : k + 1], acc_b)
        acc_c = jnp.where(m, c[k : k + 1], acc_c)
        acc_d = jnp.where(m, d[k : k + 1], acc_d)
        acc_w = jnp.where(m, cw[k : k + 1], acc_w)

    s = x - acc_w
    s2 = s * s
    s3 = s2 * s
    p = acc_a * s3 + acc_b * s2 + acc_c * s + acc_d
    out_ref[...] = jnp.clip(p, 0.0, 1.0)
    deriv = 3.0 * acc_a * s2 + 2.0 * acc_b * s + acc_c
    lad = jnp.log(jnp.abs(deriv))
    lad_ref[...] = jnp.sum(lad, axis=1, keepdims=True)


@functools.partial(jax.jit, static_argnames=("interpret",))
def kernel(inputs, unnormalized_widths, unnormalized_heights,
           unnorm_derivatives_left, unnorm_derivatives_right, *,
           interpret=False):
    B, D = inputs.shape
    nb = unnormalized_widths.shape[-1]
    uwt = unnormalized_widths.T
    uht = unnormalized_heights.T
    udlt = unnorm_derivatives_left.T
    udrt = unnorm_derivatives_right.T

    BB = 512
    grid = (B // BB,)
    full = lambda shape: pl.BlockSpec(shape, lambda i: (0, 0))
    out, lad = pl.pallas_call(
        _body,
        grid=grid,
        in_specs=[
            pl.BlockSpec((BB, D), lambda i: (i, 0)),
            full((nb, D)),
            full((nb, D)),
            full((1, D)),
            full((1, D)),
        ],
        out_specs=[
            pl.BlockSpec((BB, D), lambda i: (i, 0)),
            pl.BlockSpec((BB, 1), lambda i: (i, 0)),
        ],
        out_shape=[
            jax.ShapeDtypeStruct((B, D), jnp.float32),
            jax.ShapeDtypeStruct((B, 1), jnp.float32),
        ],
        interpret=interpret,
    )(inputs, uwt, uht, udlt, udrt)
    return out, lad.reshape(B)
